# baseline (device time: 26595 ns/iter reference)
import jax
import jax.numpy as jnp
from jax import lax
from jax.experimental import pallas as pl
from jax.experimental.pallas import tpu as pltpu

N_DEV = 16


def kernel(x, w_mat):
    m_glob, k_per = x.shape
    k_glob, n = w_mat.shape
    m_per = m_glob // N_DEV
    assert k_glob // N_DEV == k_per

    def body(x_ref, w_ref, out_ref, comm_ref, send_sems, recv_sems):
        my = lax.axis_index("i")

        barrier_sem = pltpu.get_barrier_semaphore()
        for d in range(N_DEV):
            pl.semaphore_signal(
                barrier_sem, inc=1,
                device_id=(d,), device_id_type=pl.DeviceIdType.MESH,
            )
        pl.semaphore_wait(barrier_sem, N_DEV)

        comm_ref[my] = x_ref[pl.ds(my * m_per, m_per), :]

        def desc(d, slot_dst, slot_sem):
            return pltpu.make_async_remote_copy(
                src_ref=x_ref.at[pl.ds(d * m_per, m_per), :],
                dst_ref=comm_ref.at[slot_dst],
                send_sem=send_sems.at[d],
                recv_sem=recv_sems.at[slot_sem],
                device_id=(d,),
                device_id_type=pl.DeviceIdType.MESH,
            )

        for d in range(N_DEV):
            @pl.when(my != d)
            def _(d=d):
                desc(d, my, my).start()

        for j in range(N_DEV):
            @pl.when(my != j)
            def _(j=j):
                desc(j, j, j).wait_recv()

        for d in range(N_DEV):
            @pl.when(my != d)
            def _(d=d):
                desc(d, my, my).wait_send()

        acc = jnp.zeros((m_per, n), jnp.float32)
        for j in range(N_DEV):
            acc = acc + jax.lax.dot(
                comm_ref[j],
                w_ref[pl.ds(j * k_per, k_per), :],
                preferred_element_type=jnp.float32,
            )
        c = 0.7978845608028654
        out_ref[:, :] = 0.5 * acc * (
            1.0 + jnp.tanh(c * (acc + 0.044715 * acc * acc * acc))
        )

    return pl.pallas_call(
        body,
        out_shape=jax.ShapeDtypeStruct((m_per, n), jnp.float32),
        in_specs=[
            pl.BlockSpec(memory_space=pltpu.VMEM),
            pl.BlockSpec(memory_space=pltpu.VMEM),
        ],
        out_specs=pl.BlockSpec(memory_space=pltpu.VMEM),
        scratch_shapes=[
            pltpu.VMEM((N_DEV, m_per, k_per), x.dtype),
            pltpu.SemaphoreType.DMA((N_DEV,)),
            pltpu.SemaphoreType.DMA((N_DEV,)),
        ],
        compiler_params=pltpu.CompilerParams(collective_id=0),
    )(x, w_mat)


# device time: 25157 ns/iter; 1.0572x vs baseline; 1.0572x over previous
import jax
import jax.numpy as jnp
from jax import lax
from jax.experimental import pallas as pl
from jax.experimental.pallas import tpu as pltpu

N_DEV = 16


def kernel(x, w_mat):
    m_glob, k_per = x.shape
    k_glob, n = w_mat.shape
    m_per = m_glob // N_DEV
    assert k_glob // N_DEV == k_per

    def body(x_ref, w_ref, out_ref, xbf, comm, wbuf,
             send_sems, recv_sems, wsems):
        my = lax.axis_index("i")

        for j in range(N_DEV):
            pltpu.make_async_copy(
                w_ref.at[pl.ds(j * k_per, k_per), :], wbuf.at[j], wsems.at[j]
            ).start()
        xbf[:, :] = x_ref[:, :].astype(jnp.bfloat16)
        comm[my] = xbf[pl.ds(my * m_per, m_per), :]

        barrier_sem = pltpu.get_barrier_semaphore()
        for k in range(4):
            partner = lax.rem(my + (1 << k), N_DEV)
            pl.semaphore_signal(
                barrier_sem, inc=1,
                device_id=(partner,), device_id_type=pl.DeviceIdType.MESH,
            )
            pl.semaphore_wait(barrier_sem, 1)

        def desc(d, slot_dst, slot_sem):
            return pltpu.make_async_remote_copy(
                src_ref=xbf.at[pl.ds(d * m_per, m_per), :],
                dst_ref=comm.at[slot_dst],
                send_sem=send_sems.at[d],
                recv_sem=recv_sems.at[slot_sem],
                device_id=(d,), device_id_type=pl.DeviceIdType.MESH,
            )

        for k in range(1, N_DEV):
            d = lax.rem(my + k, N_DEV)
            desc(d, my, my).start()

        acc = jnp.zeros((m_per, n), jnp.float32)
        for j in range(N_DEV):
            pltpu.make_async_copy(
                w_ref.at[pl.ds(j * k_per, k_per), :], wbuf.at[j], wsems.at[j]
            ).wait()
            @pl.when(my != j)
            def _(j=j):
                desc(j, j, j).wait_recv()
            acc = acc + jax.lax.dot(
                comm[j].astype(jnp.float32), wbuf[j],
                preferred_element_type=jnp.float32,
            )
        c = 0.7978845608028654
        out_ref[:, :] = 0.5 * acc * (
            1.0 + jnp.tanh(c * (acc + 0.044715 * acc * acc * acc))
        )
        for k in range(1, N_DEV):
            d = lax.rem(my + k, N_DEV)
            desc(d, my, my).wait_send()

    return pl.pallas_call(
        body,
        out_shape=jax.ShapeDtypeStruct((m_per, n), jnp.float32),
        in_specs=[
            pl.BlockSpec(memory_space=pltpu.VMEM),
            pl.BlockSpec(memory_space=pl.ANY),
        ],
        out_specs=pl.BlockSpec(memory_space=pltpu.VMEM),
        scratch_shapes=[
            pltpu.VMEM((k_glob, k_per), jnp.bfloat16),
            pltpu.VMEM((N_DEV, m_per, k_per), jnp.bfloat16),
            pltpu.VMEM((N_DEV, k_per, n), jnp.float32),
            pltpu.SemaphoreType.DMA((N_DEV,)),
            pltpu.SemaphoreType.DMA((N_DEV,)),
            pltpu.SemaphoreType.DMA((N_DEV,)),
        ],
        compiler_params=pltpu.CompilerParams(collective_id=0),
    )(x, w_mat)


# device time: 25077 ns/iter; 1.0605x vs baseline; 1.0032x over previous
import jax
import jax.numpy as jnp
from jax import lax
from jax.experimental import pallas as pl
from jax.experimental.pallas import tpu as pltpu

N_DEV = 16


def kernel(x, w_mat):
    m_glob, k_per = x.shape
    k_glob, n = w_mat.shape
    m_per = m_glob // N_DEV
    assert k_glob // N_DEV == k_per

    def body(x_ref, w_ref, out_ref, xbf, comm, wbuf,
             send_sems, recv_sems, wsems):
        my = lax.axis_index("i")

        for j in range(N_DEV):
            pltpu.make_async_copy(
                w_ref.at[pl.ds(j * k_per, k_per), :], wbuf.at[j], wsems.at[j]
            ).start()
        xbf[:, :] = x_ref[:, :].astype(jnp.bfloat16)
        comm[my] = xbf[pl.ds(my * m_per, m_per), :]

        barrier_sem = pltpu.get_barrier_semaphore()
        for k in range(4):
            partner = lax.rem(my + (1 << k), N_DEV)
            pl.semaphore_signal(
                barrier_sem, inc=1,
                device_id=(partner,), device_id_type=pl.DeviceIdType.MESH,
            )
            pl.semaphore_wait(barrier_sem, 1)

        def desc(d, slot_dst, slot_sem):
            return pltpu.make_async_remote_copy(
                src_ref=xbf.at[pl.ds(d * m_per, m_per), :],
                dst_ref=comm.at[slot_dst],
                send_sem=send_sems.at[d],
                recv_sem=recv_sems.at[slot_sem],
                device_id=(d,), device_id_type=pl.DeviceIdType.MESH,
            )

        for k in range(1, N_DEV):
            d = lax.rem(my + k, N_DEV)
            desc(d, my, my).start()

        acc = jnp.zeros((m_per, n), jnp.float32)
        for k in range(N_DEV):
            j = lax.rem(my + (N_DEV - k), N_DEV) if k else my
            pltpu.make_async_copy(
                w_ref.at[pl.ds(j * k_per, k_per), :], wbuf.at[j], wsems.at[j]
            ).wait()
            if k:
                desc(j, j, j).wait_recv()
            acc = acc + jax.lax.dot(
                comm[j].astype(jnp.float32), wbuf[j],
                preferred_element_type=jnp.float32,
            )
        c = 0.7978845608028654
        out_ref[:, :] = 0.5 * acc * (
            1.0 + jnp.tanh(c * (acc + 0.044715 * acc * acc * acc))
        )
        for k in range(1, N_DEV):
            d = lax.rem(my + k, N_DEV)
            desc(d, my, my).wait_send()

    return pl.pallas_call(
        body,
        out_shape=jax.ShapeDtypeStruct((m_per, n), jnp.float32),
        in_specs=[
            pl.BlockSpec(memory_space=pltpu.VMEM),
            pl.BlockSpec(memory_space=pl.ANY),
        ],
        out_specs=pl.BlockSpec(memory_space=pltpu.VMEM),
        scratch_shapes=[
            pltpu.VMEM((k_glob, k_per), jnp.bfloat16),
            pltpu.VMEM((N_DEV, m_per, k_per), jnp.bfloat16),
            pltpu.VMEM((N_DEV, k_per, n), jnp.float32),
            pltpu.SemaphoreType.DMA((N_DEV,)),
            pltpu.SemaphoreType.DMA((N_DEV,)),
            pltpu.SemaphoreType.DMA((N_DEV,)),
        ],
        compiler_params=pltpu.CompilerParams(collective_id=0),
    )(x, w_mat)


# device time: 23049 ns/iter; 1.1538x vs baseline; 1.0880x over previous
import jax
import jax.numpy as jnp
from jax import lax
from jax.experimental import pallas as pl
from jax.experimental.pallas import tpu as pltpu

N_DEV = 16


def kernel(x, w_mat):
    m_glob, k_per = x.shape
    k_glob, n = w_mat.shape
    m_per = m_glob // N_DEV
    assert k_glob // N_DEV == k_per

    def body(x_ref, w_ref, out_ref, xbf, comm, wbuf,
             send_sems, recv_sems, wsems):
        my = lax.axis_index("i")

        for j in range(N_DEV):
            pltpu.make_async_copy(
                w_ref.at[pl.ds(j * k_per, k_per), :], wbuf.at[j], wsems.at[j]
            ).start()
        xbf[:, :] = x_ref[:, :].astype(jnp.bfloat16)
        comm[my] = xbf[pl.ds(my * m_per, m_per), :]

        barrier_sem = pltpu.get_barrier_semaphore()
        for d in range(N_DEV):
            pl.semaphore_signal(
                barrier_sem, inc=1,
                device_id=(d,), device_id_type=pl.DeviceIdType.MESH,
            )
        pl.semaphore_wait(barrier_sem, N_DEV)

        def desc(d, slot_dst, slot_sem):
            return pltpu.make_async_remote_copy(
                src_ref=xbf.at[pl.ds(d * m_per, m_per), :],
                dst_ref=comm.at[slot_dst],
                send_sem=send_sems.at[d],
                recv_sem=recv_sems.at[slot_sem],
                device_id=(d,), device_id_type=pl.DeviceIdType.MESH,
            )

        for k in range(1, N_DEV):
            d = lax.rem(my + k, N_DEV)
            desc(d, my, my).start()

        acc = jnp.zeros((m_per, n), jnp.float32)
        for k in range(N_DEV):
            j = lax.rem(my + (N_DEV - k), N_DEV) if k else my
            pltpu.make_async_copy(
                w_ref.at[pl.ds(j * k_per, k_per), :], wbuf.at[j], wsems.at[j]
            ).wait()
            if k:
                desc(j, j, j).wait_recv()
            acc = acc + jax.lax.dot(
                comm[j].astype(jnp.float32), wbuf[j],
                preferred_element_type=jnp.float32,
            )
        c = 0.7978845608028654
        out_ref[:, :] = 0.5 * acc * (
            1.0 + jnp.tanh(c * (acc + 0.044715 * acc * acc * acc))
        )
        for k in range(1, N_DEV):
            d = lax.rem(my + k, N_DEV)
            desc(d, my, my).wait_send()

    return pl.pallas_call(
        body,
        out_shape=jax.ShapeDtypeStruct((m_per, n), jnp.float32),
        in_specs=[
            pl.BlockSpec(memory_space=pltpu.VMEM),
            pl.BlockSpec(memory_space=pl.ANY),
        ],
        out_specs=pl.BlockSpec(memory_space=pltpu.VMEM),
        scratch_shapes=[
            pltpu.VMEM((k_glob, k_per), jnp.bfloat16),
            pltpu.VMEM((N_DEV, m_per, k_per), jnp.bfloat16),
            pltpu.VMEM((N_DEV, k_per, n), jnp.float32),
            pltpu.SemaphoreType.DMA((N_DEV,)),
            pltpu.SemaphoreType.DMA((N_DEV,)),
            pltpu.SemaphoreType.DMA((N_DEV,)),
        ],
        compiler_params=pltpu.CompilerParams(collective_id=0),
    )(x, w_mat)
